# baseline (speedup 1.0000x reference)
import jax
import jax.numpy as jnp
from jax import lax
from jax.experimental import pallas as pl
from jax.experimental.pallas import tpu as pltpu
from jax.experimental.pallas import tpu_sc as plsc

N = 10000
E = 320000
D = 128
DE = 16

NC = 2
NS = 16
NW = NC * NS
EPW = E // NW
CH = 40
NCHUNK = EPW // CH
SROW = 624
CPY = 24
TAILB = NS * SROW
TAIL = N - TAILB


def _node_pre_body(h_ref, pwh_ref, wu_ref, wv_ref, z_ref, up_ref, vp_ref):
    h = h_ref[...]
    z_ref[...] = jnp.dot(h, pwh_ref[...], preferred_element_type=jnp.float32)
    up_ref[:, :DE] = jnp.dot(h, wu_ref[...], preferred_element_type=jnp.float32)
    up_ref[:, DE:] = jnp.zeros((h.shape[0], D - DE), jnp.float32)
    vp_ref[:, :DE] = jnp.dot(h, wv_ref[...], preferred_element_type=jnp.float32)
    vp_ref[:, DE:] = jnp.zeros((h.shape[0], D - DE), jnp.float32)


def _edge_pre_body(e_ref, pwe_ref, pb_ref, wwe_ref, wb_ref, me_ref, te_ref):
    ev = e_ref[...]
    me_ref[...] = jnp.dot(ev, pwe_ref[...], preferred_element_type=jnp.float32) + pb_ref[...]
    te_ref[...] = jnp.dot(ev, wwe_ref[...], preferred_element_type=jnp.float32) + wb_ref[...]


def _node_post_body(h_ref, agg_ref, deg_ref, qh_ref, qa_ref, qb_ref, o_ref):
    agg = agg_ref[0] + agg_ref[1]
    deg = deg_ref[0][:, 0:1] + deg_ref[1][:, 0:1]
    aggn = agg / deg
    acc = (jnp.dot(h_ref[...], qh_ref[...], preferred_element_type=jnp.float32)
           + jnp.dot(aggn, qa_ref[...], preferred_element_type=jnp.float32)
           + qb_ref[...])
    o_ref[...] = jnp.maximum(acc, 0.0)


def _sc_body(z_hbm, me_hbm, te_hbm, up_hbm, vp_hbm, src_hbm, tgt_hbm,
             zro_hbm, one_hbm,
             agg_out, deg_out, enew_out,
             sidx, tidx, zbuf, mebuf, tebuf, ubuf, vbuf, zb128,
             agg_sp, sem):
    c = lax.axis_index("c")
    s = lax.axis_index("s")
    wid = s * NC + c

    def zero_accum():
        for j in range(SROW // CPY):
            pltpu.sync_copy(zb128, agg_sp.at[pl.ds(s * SROW + j * CPY, CPY)])

        @pl.when(s == NS - 1)
        def _zero_tail():
            pltpu.sync_copy(zb128.at[pl.ds(0, TAIL)], agg_sp.at[pl.ds(TAILB, TAIL)])

    def copy_accum_out(out_ref):
        for j in range(SROW // CPY):
            pltpu.sync_copy(agg_sp.at[pl.ds(s * SROW + j * CPY, CPY)], zb128)
            pltpu.sync_copy(zb128, out_ref.at[c, pl.ds(s * SROW + j * CPY, CPY)])

        @pl.when(s == NS - 1)
        def _copy_tail():
            pltpu.sync_copy(agg_sp.at[pl.ds(TAILB, TAIL)], zb128.at[pl.ds(0, TAIL)])
            pltpu.sync_copy(zb128.at[pl.ds(0, TAIL)], out_ref.at[c, pl.ds(TAILB, TAIL)])

    pltpu.sync_copy(zro_hbm, zb128)
    zero_accum()
    plsc.subcore_barrier()

    def chunk(i, carry):
        base = wid * EPW + i * CH
        pltpu.sync_copy(src_hbm.at[pl.ds(base, CH)], sidx)
        pltpu.sync_copy(tgt_hbm.at[pl.ds(base, CH)], tidx)
        pltpu.async_copy(z_hbm.at[sidx], zbuf, sem).wait()
        pltpu.sync_copy(me_hbm.at[pl.ds(base, CH)], mebuf)

        def relu_add(k, carry2):
            r = k >> 3
            col = (k & 7) * 16
            mv = mebuf[r, pl.ds(col, 16)]
            zv = zbuf[r, pl.ds(col, 16)]
            mebuf[r, pl.ds(col, 16)] = jnp.maximum(mv + zv, 0.0)
            return carry2
        lax.fori_loop(0, CH * (D // 16), relu_add, 0, unroll=4)

        pltpu.sync_copy(mebuf, agg_sp.at[tidx], add=True)

        pltpu.async_copy(up_hbm.at[sidx], ubuf, sem).wait()
        pltpu.async_copy(vp_hbm.at[tidx], vbuf, sem).wait()
        pltpu.sync_copy(te_hbm.at[pl.ds(base, CH)], tebuf)

        def enew(r, carry2):
            tv = tebuf[r, pl.ds(0, 16)]
            uv = ubuf[r, pl.ds(0, 16)]
            vv = vbuf[r, pl.ds(0, 16)]
            tebuf[r, pl.ds(0, 16)] = jnp.maximum(tv + uv + vv, 0.0)
            return carry2
        lax.fori_loop(0, CH, enew, 0, unroll=4)
        pltpu.sync_copy(tebuf, enew_out.at[pl.ds(base, CH)])
        return carry

    lax.fori_loop(0, NCHUNK, chunk, 0)
    plsc.subcore_barrier()
    copy_accum_out(agg_out)
    plsc.subcore_barrier()

    pltpu.sync_copy(zro_hbm, zb128)
    zero_accum()
    pltpu.sync_copy(one_hbm, mebuf)
    plsc.subcore_barrier()

    def deg_chunk(i, carry):
        base = wid * EPW + i * CH
        pltpu.sync_copy(tgt_hbm.at[pl.ds(base, CH)], tidx)
        pltpu.sync_copy(mebuf, agg_sp.at[tidx], add=True)
        return carry

    lax.fori_loop(0, NCHUNK, deg_chunk, 0)
    plsc.subcore_barrier()
    copy_accum_out(deg_out)


def _sc_edge_stage(Z, Me, Te, Up, Vp, src, tgt):
    zro = jnp.zeros((CPY, D), jnp.float32)
    one = jnp.ones((CH, D), jnp.float32)
    mesh = plsc.VectorSubcoreMesh(core_axis_name="c", subcore_axis_name="s")
    fn = pl.kernel(
        _sc_body,
        out_type=(
            jax.ShapeDtypeStruct((NC, N, D), jnp.float32),
            jax.ShapeDtypeStruct((NC, N, D), jnp.float32),
            jax.ShapeDtypeStruct((E, DE), jnp.float32),
        ),
        mesh=mesh,
        scratch_types=[
            pltpu.VMEM((CH,), jnp.int32),
            pltpu.VMEM((CH,), jnp.int32),
            pltpu.VMEM((CH, D), jnp.float32),
            pltpu.VMEM((CH, D), jnp.float32),
            pltpu.VMEM((CH, DE), jnp.float32),
            pltpu.VMEM((CH, D), jnp.float32),
            pltpu.VMEM((CH, D), jnp.float32),
            pltpu.VMEM((CPY, D), jnp.float32),
            pltpu.VMEM_SHARED((N, D), jnp.float32),
            pltpu.SemaphoreType.DMA,
        ],
    )
    return fn(Z, Me, Te, Up, Vp, src, tgt, zro, one)


def kernel(h, e, edge_index, P_w, P_b, Q_w, Q_b, W_w, W_b):
    src = edge_index[0]
    tgt = edge_index[1]

    nb = N // 1000
    Z, Up, Vp = pl.pallas_call(
        _node_pre_body,
        grid=(nb,),
        in_specs=[
            pl.BlockSpec((1000, D), lambda i: (i, 0)),
            pl.BlockSpec((D, D), lambda i: (0, 0)),
            pl.BlockSpec((D, DE), lambda i: (0, 0)),
            pl.BlockSpec((D, DE), lambda i: (0, 0)),
        ],
        out_specs=[
            pl.BlockSpec((1000, D), lambda i: (i, 0)),
            pl.BlockSpec((1000, D), lambda i: (i, 0)),
            pl.BlockSpec((1000, D), lambda i: (i, 0)),
        ],
        out_shape=[
            jax.ShapeDtypeStruct((N, D), jnp.float32),
            jax.ShapeDtypeStruct((N, D), jnp.float32),
            jax.ShapeDtypeStruct((N, D), jnp.float32),
        ],
    )(h, P_w[:D], W_w[DE:DE + D], W_w[DE + D:])

    eb = E // 2000
    Me, Te = pl.pallas_call(
        _edge_pre_body,
        grid=(eb,),
        in_specs=[
            pl.BlockSpec((2000, DE), lambda i: (i, 0)),
            pl.BlockSpec((DE, D), lambda i: (0, 0)),
            pl.BlockSpec((1, D), lambda i: (0, 0)),
            pl.BlockSpec((DE, DE), lambda i: (0, 0)),
            pl.BlockSpec((1, DE), lambda i: (0, 0)),
        ],
        out_specs=[
            pl.BlockSpec((2000, D), lambda i: (i, 0)),
            pl.BlockSpec((2000, DE), lambda i: (i, 0)),
        ],
        out_shape=[
            jax.ShapeDtypeStruct((E, D), jnp.float32),
            jax.ShapeDtypeStruct((E, DE), jnp.float32),
        ],
    )(e, P_w[D:], P_b.reshape(1, D), W_w[:DE], W_b.reshape(1, DE))

    agg, degs, e_new = _sc_edge_stage(Z, Me, Te, Up, Vp, src, tgt)

    h_new = pl.pallas_call(
        _node_post_body,
        grid=(nb,),
        in_specs=[
            pl.BlockSpec((1000, D), lambda i: (i, 0)),
            pl.BlockSpec((NC, 1000, D), lambda i: (0, i, 0)),
            pl.BlockSpec((NC, 1000, D), lambda i: (0, i, 0)),
            pl.BlockSpec((D, D), lambda i: (0, 0)),
            pl.BlockSpec((D, D), lambda i: (0, 0)),
            pl.BlockSpec((1, D), lambda i: (0, 0)),
        ],
        out_specs=pl.BlockSpec((1000, D), lambda i: (i, 0)),
        out_shape=jax.ShapeDtypeStruct((N, D), jnp.float32),
    )(h, agg, degs, Q_w[:D], Q_w[D:], Q_b.reshape(1, D))

    return (h_new, e_new)


# speedup vs baseline: 1.5550x; 1.5550x over previous
import jax
import jax.numpy as jnp
from jax import lax
from jax.experimental import pallas as pl
from jax.experimental.pallas import tpu as pltpu
from jax.experimental.pallas import tpu_sc as plsc

N = 10000
E = 320000
D = 128
DE = 16

NC = 2
NS = 16
NW = NC * NS
EPW = E // NW
CH = 40
NCHUNK = EPW // CH
SUP = 5
NBLK = NCHUNK // SUP
SROW = 624
CPY = 16
TAILB = NS * SROW
TAIL = N - TAILB


def _node_pre_body(h_ref, pwh_ref, wu_ref, wv_ref, z_ref, up_ref, vp_ref):
    h = h_ref[...]
    z_ref[...] = jnp.dot(h, pwh_ref[...], preferred_element_type=jnp.float32)
    up_ref[:, :DE] = jnp.dot(h, wu_ref[...], preferred_element_type=jnp.float32)
    up_ref[:, DE:] = jnp.zeros((h.shape[0], D - DE), jnp.float32)
    vp_ref[:, :DE] = jnp.dot(h, wv_ref[...], preferred_element_type=jnp.float32)
    vp_ref[:, DE:] = jnp.zeros((h.shape[0], D - DE), jnp.float32)


def _edge_pre_body(e_ref, pwe_ref, pb_ref, wwe_ref, wb_ref, me_ref, te_ref):
    ev = e_ref[...]
    me_ref[...] = jnp.dot(ev, pwe_ref[...], preferred_element_type=jnp.float32) + pb_ref[...]
    te_ref[...] = jnp.dot(ev, wwe_ref[...], preferred_element_type=jnp.float32) + wb_ref[...]


def _node_post_body(h_ref, agg_ref, deg_ref, qh_ref, qa_ref, qb_ref, o_ref):
    agg = agg_ref[0] + agg_ref[1]
    deg = deg_ref[0][:, 0:1] + deg_ref[1][:, 0:1]
    aggn = agg / deg
    acc = (jnp.dot(h_ref[...], qh_ref[...], preferred_element_type=jnp.float32)
           + jnp.dot(aggn, qa_ref[...], preferred_element_type=jnp.float32)
           + qb_ref[...])
    o_ref[...] = jnp.maximum(acc, 0.0)


def _sc_body(z_hbm, me_hbm, te_hbm, up_hbm, vp_hbm, src3_hbm, tgt3_hbm,
             zro_hbm, one_hbm,
             agg_out, deg_out, enew_out,
             sidxs, tidxs, zbufA, mebufA, ubufA, vbufA,
             zbufB, mebufB, ubufB, vbufB, tebuf, zb128,
             agg_sp, semGA, semLA, semGB, semLB, semD):
    c = lax.axis_index("c")
    s = lax.axis_index("s")
    wid = s * NC + c

    def zero_accum():
        for j in range(SROW // CPY):
            pltpu.sync_copy(zb128, agg_sp.at[pl.ds(s * SROW + j * CPY, CPY)])

        @pl.when(s == NS - 1)
        def _zero_tail():
            pltpu.sync_copy(zb128.at[pl.ds(0, TAIL)], agg_sp.at[pl.ds(TAILB, TAIL)])

    def copy_accum_out(out_ref):
        for j in range(SROW // CPY):
            pltpu.sync_copy(agg_sp.at[pl.ds(s * SROW + j * CPY, CPY)], zb128)
            pltpu.sync_copy(zb128, out_ref.at[c, pl.ds(s * SROW + j * CPY, CPY)])

        @pl.when(s == NS - 1)
        def _copy_tail():
            pltpu.sync_copy(agg_sp.at[pl.ds(TAILB, TAIL)], zb128.at[pl.ds(0, TAIL)])
            pltpu.sync_copy(zb128.at[pl.ds(0, TAIL)], out_ref.at[c, pl.ds(TAILB, TAIL)])

    def load_idx(b):
        pltpu.sync_copy(src3_hbm.at[wid, b], sidxs)
        pltpu.sync_copy(tgt3_hbm.at[wid, b], tidxs)

    def fire(c1, zb, mb, ub, vb, semG, semL):
        row = lax.rem(c1, SUP)
        base = wid * EPW + c1 * CH
        pltpu.async_copy(z_hbm.at[sidxs.at[row]], zb, semG)
        pltpu.async_copy(up_hbm.at[sidxs.at[row]], ub, semG)
        pltpu.async_copy(vp_hbm.at[tidxs.at[row]], vb, semG)
        pltpu.async_copy(me_hbm.at[pl.ds(base, CH)], mb, semL)

    def process(ci, zb, mb, ub, vb, semG, semL, zb2, mb2, ub2, vb2, semG2, semL2):
        row = lax.rem(ci, SUP)
        base = wid * EPW + ci * CH
        c1 = ci + 1

        @pl.when(jnp.logical_and(c1 < NCHUNK, lax.rem(c1, SUP) != 0))
        def _prefetch_early():
            fire(c1, zb2, mb2, ub2, vb2, semG2, semL2)

        pltpu.make_async_copy(z_hbm.at[sidxs.at[row]], zb, semG).wait()
        pltpu.make_async_copy(up_hbm.at[sidxs.at[row]], ub, semG).wait()
        pltpu.make_async_copy(vp_hbm.at[tidxs.at[row]], vb, semG).wait()
        pltpu.make_async_copy(me_hbm.at[pl.ds(base, CH)], mb, semL).wait()

        def relu_add(k, carry2):
            r = k >> 3
            col = (k & 7) * 16
            mv = mb[r, pl.ds(col, 16)]
            zv = zb[r, pl.ds(col, 16)]
            mb[r, pl.ds(col, 16)] = jnp.maximum(mv + zv, 0.0)
            return carry2
        lax.fori_loop(0, CH * (D // 16), relu_add, 0, unroll=4)

        pltpu.sync_copy(mb, agg_sp.at[tidxs.at[row]], add=True)

        pltpu.sync_copy(te_hbm.at[pl.ds(base, CH)], tebuf)

        def enew(r, carry2):
            tv = tebuf[r, pl.ds(0, 16)]
            uv = ub[r, pl.ds(0, 16)]
            vv = vb[r, pl.ds(0, 16)]
            tebuf[r, pl.ds(0, 16)] = jnp.maximum(tv + uv + vv, 0.0)
            return carry2
        lax.fori_loop(0, CH, enew, 0, unroll=4)
        pltpu.sync_copy(tebuf, enew_out.at[pl.ds(base, CH)])

        @pl.when(jnp.logical_and(c1 < NCHUNK, lax.rem(c1, SUP) == 0))
        def _prefetch_boundary():
            load_idx(c1 // SUP)
            fire(c1, zb2, mb2, ub2, vb2, semG2, semL2)

    pltpu.sync_copy(zro_hbm, zb128)
    zero_accum()
    plsc.subcore_barrier()

    load_idx(0)
    fire(0, zbufA, mebufA, ubufA, vbufA, semGA, semLA)

    def pair(i2, carry):
        ci = 2 * i2
        process(ci, zbufA, mebufA, ubufA, vbufA, semGA, semLA,
                zbufB, mebufB, ubufB, vbufB, semGB, semLB)
        process(ci + 1, zbufB, mebufB, ubufB, vbufB, semGB, semLB,
                zbufA, mebufA, ubufA, vbufA, semGA, semLA)
        return carry

    lax.fori_loop(0, NCHUNK // 2, pair, 0)
    plsc.subcore_barrier()
    copy_accum_out(agg_out)
    plsc.subcore_barrier()

    pltpu.sync_copy(zro_hbm, zb128)
    zero_accum()
    pltpu.sync_copy(one_hbm, mebufA)
    plsc.subcore_barrier()

    def deg_block(b, carry):
        load_idx(b)
        for k in range(SUP):
            pltpu.async_copy(mebufA, agg_sp.at[tidxs.at[k]], semD, add=True)
        for k in range(SUP):
            pltpu.make_async_copy(mebufA, agg_sp.at[tidxs.at[k]], semD).wait()
        return carry

    lax.fori_loop(0, NBLK, deg_block, 0)
    plsc.subcore_barrier()
    copy_accum_out(deg_out)


def _sc_edge_stage(Z, Me, Te, Up, Vp, src, tgt):
    zro = jnp.zeros((CPY, D), jnp.float32)
    one = jnp.ones((CH, D), jnp.float32)
    mesh = plsc.VectorSubcoreMesh(core_axis_name="c", subcore_axis_name="s")
    fn = pl.kernel(
        _sc_body,
        out_type=(
            jax.ShapeDtypeStruct((NC, N, D), jnp.float32),
            jax.ShapeDtypeStruct((NC, N, D), jnp.float32),
            jax.ShapeDtypeStruct((E, DE), jnp.float32),
        ),
        mesh=mesh,
        scratch_types=[
            pltpu.VMEM((SUP, CH), jnp.int32),
            pltpu.VMEM((SUP, CH), jnp.int32),
            pltpu.VMEM((CH, D), jnp.float32),
            pltpu.VMEM((CH, D), jnp.float32),
            pltpu.VMEM((CH, D), jnp.float32),
            pltpu.VMEM((CH, D), jnp.float32),
            pltpu.VMEM((CH, D), jnp.float32),
            pltpu.VMEM((CH, D), jnp.float32),
            pltpu.VMEM((CH, D), jnp.float32),
            pltpu.VMEM((CH, D), jnp.float32),
            pltpu.VMEM((CH, DE), jnp.float32),
            pltpu.VMEM((CPY, D), jnp.float32),
            pltpu.VMEM_SHARED((N, D), jnp.float32),
            pltpu.SemaphoreType.DMA,
            pltpu.SemaphoreType.DMA,
            pltpu.SemaphoreType.DMA,
            pltpu.SemaphoreType.DMA,
            pltpu.SemaphoreType.DMA,
        ],
    )
    src3 = src.reshape(NW, NBLK, SUP, CH)
    tgt3 = tgt.reshape(NW, NBLK, SUP, CH)
    return fn(Z, Me, Te, Up, Vp, src3, tgt3, zro, one)


def kernel(h, e, edge_index, P_w, P_b, Q_w, Q_b, W_w, W_b):
    src = edge_index[0]
    tgt = edge_index[1]

    nb = N // 1000
    Z, Up, Vp = pl.pallas_call(
        _node_pre_body,
        grid=(nb,),
        in_specs=[
            pl.BlockSpec((1000, D), lambda i: (i, 0)),
            pl.BlockSpec((D, D), lambda i: (0, 0)),
            pl.BlockSpec((D, DE), lambda i: (0, 0)),
            pl.BlockSpec((D, DE), lambda i: (0, 0)),
        ],
        out_specs=[
            pl.BlockSpec((1000, D), lambda i: (i, 0)),
            pl.BlockSpec((1000, D), lambda i: (i, 0)),
            pl.BlockSpec((1000, D), lambda i: (i, 0)),
        ],
        out_shape=[
            jax.ShapeDtypeStruct((N, D), jnp.float32),
            jax.ShapeDtypeStruct((N, D), jnp.float32),
            jax.ShapeDtypeStruct((N, D), jnp.float32),
        ],
    )(h, P_w[:D], W_w[DE:DE + D], W_w[DE + D:])

    eb = E // 2000
    Me, Te = pl.pallas_call(
        _edge_pre_body,
        grid=(eb,),
        in_specs=[
            pl.BlockSpec((2000, DE), lambda i: (i, 0)),
            pl.BlockSpec((DE, D), lambda i: (0, 0)),
            pl.BlockSpec((1, D), lambda i: (0, 0)),
            pl.BlockSpec((DE, DE), lambda i: (0, 0)),
            pl.BlockSpec((1, DE), lambda i: (0, 0)),
        ],
        out_specs=[
            pl.BlockSpec((2000, D), lambda i: (i, 0)),
            pl.BlockSpec((2000, DE), lambda i: (i, 0)),
        ],
        out_shape=[
            jax.ShapeDtypeStruct((E, D), jnp.float32),
            jax.ShapeDtypeStruct((E, DE), jnp.float32),
        ],
    )(e, P_w[D:], P_b.reshape(1, D), W_w[:DE], W_b.reshape(1, DE))

    agg, degs, e_new = _sc_edge_stage(Z, Me, Te, Up, Vp, src, tgt)

    h_new = pl.pallas_call(
        _node_post_body,
        grid=(nb,),
        in_specs=[
            pl.BlockSpec((1000, D), lambda i: (i, 0)),
            pl.BlockSpec((NC, 1000, D), lambda i: (0, i, 0)),
            pl.BlockSpec((NC, 1000, D), lambda i: (0, i, 0)),
            pl.BlockSpec((D, D), lambda i: (0, 0)),
            pl.BlockSpec((D, D), lambda i: (0, 0)),
            pl.BlockSpec((1, D), lambda i: (0, 0)),
        ],
        out_specs=pl.BlockSpec((1000, D), lambda i: (i, 0)),
        out_shape=jax.ShapeDtypeStruct((N, D), jnp.float32),
    )(h, agg, degs, Q_w[:D], Q_w[D:], Q_b.reshape(1, D))

    return (h_new, e_new)


# speedup vs baseline: 1.7434x; 1.1212x over previous
import jax
import jax.numpy as jnp
from jax import lax
from jax.experimental import pallas as pl
from jax.experimental.pallas import tpu as pltpu
from jax.experimental.pallas import tpu_sc as plsc

N = 10000
E = 320000
D = 128
DE = 16

NC = 2
NS = 16
NW = NC * NS
EPW = E // NW
CH = 40
NCHUNK = EPW // CH
SUP = 5
NBLK = NCHUNK // SUP
SROW = 624
CPY = 16
TAILB = NS * SROW
TAIL = N - TAILB


def _node_pre_body(h_ref, pwh_ref, wu_ref, wv_ref, z_ref, up_ref, vp_ref):
    h = h_ref[...]
    z_ref[...] = jnp.dot(h, pwh_ref[...], preferred_element_type=jnp.float32)
    up_ref[:, :DE] = jnp.dot(h, wu_ref[...], preferred_element_type=jnp.float32)
    up_ref[:, DE:] = jnp.zeros((h.shape[0], D - DE), jnp.float32)
    vp_ref[:, :DE] = jnp.dot(h, wv_ref[...], preferred_element_type=jnp.float32)
    vp_ref[:, DE:] = jnp.zeros((h.shape[0], D - DE), jnp.float32)


def _edge_pre_body(e_ref, pwe_ref, pb_ref, wwe_ref, wb_ref, me_ref, te_ref):
    ev = e_ref[...]
    me_ref[...] = jnp.dot(ev, pwe_ref[...], preferred_element_type=jnp.float32) + pb_ref[...]
    te_ref[...] = jnp.dot(ev, wwe_ref[...], preferred_element_type=jnp.float32) + wb_ref[...]


def _edge_post_body(te_ref, s_ref, o_ref):
    o_ref[...] = jnp.maximum(te_ref[...] + s_ref[...], 0.0)


def _node_post_body(h_ref, agg_ref, deg_ref, qh_ref, qa_ref, qb_ref, o_ref):
    agg = agg_ref[0] + agg_ref[1]
    deg = deg_ref[0][:, 0:1] + deg_ref[1][:, 0:1]
    aggn = agg / deg
    acc = (jnp.dot(h_ref[...], qh_ref[...], preferred_element_type=jnp.float32)
           + jnp.dot(aggn, qa_ref[...], preferred_element_type=jnp.float32)
           + qb_ref[...])
    o_ref[...] = jnp.maximum(acc, 0.0)


def _sc_body(z_hbm, me_hbm, up_hbm, vp_hbm, src3_hbm, tgt3_hbm,
             zro_hbm, one_hbm,
             agg_out, deg_out, enew_out,
             sidxs, tidxs, zbufA, mebufA, ubufA, vbufA,
             zbufB, mebufB, ubufB, vbufB, sbuf, zb128,
             agg_sp, semGA, semLA, semGB, semLB, semD, semS, semT):
    c = lax.axis_index("c")
    s = lax.axis_index("s")
    wid = s * NC + c

    def zero_accum():
        for j in range(SROW // CPY):
            pltpu.sync_copy(zb128, agg_sp.at[pl.ds(s * SROW + j * CPY, CPY)])

        @pl.when(s == NS - 1)
        def _zero_tail():
            pltpu.sync_copy(zb128.at[pl.ds(0, TAIL)], agg_sp.at[pl.ds(TAILB, TAIL)])

    def copy_accum_out(out_ref):
        for j in range(SROW // CPY):
            pltpu.sync_copy(agg_sp.at[pl.ds(s * SROW + j * CPY, CPY)], zb128)
            pltpu.sync_copy(zb128, out_ref.at[c, pl.ds(s * SROW + j * CPY, CPY)])

        @pl.when(s == NS - 1)
        def _copy_tail():
            pltpu.sync_copy(agg_sp.at[pl.ds(TAILB, TAIL)], zb128.at[pl.ds(0, TAIL)])
            pltpu.sync_copy(zb128.at[pl.ds(0, TAIL)], out_ref.at[c, pl.ds(TAILB, TAIL)])

    def load_idx(b):
        pltpu.sync_copy(src3_hbm.at[wid, b], sidxs)
        pltpu.sync_copy(tgt3_hbm.at[wid, b], tidxs)

    def fire(c1, zb, mb, ub, vb, semG, semL):
        row = lax.rem(c1, SUP)
        base = wid * EPW + c1 * CH
        pltpu.async_copy(z_hbm.at[sidxs.at[row]], zb, semG)
        pltpu.async_copy(up_hbm.at[sidxs.at[row]], ub, semG)
        pltpu.async_copy(vp_hbm.at[tidxs.at[row]], vb, semG)
        pltpu.async_copy(me_hbm.at[pl.ds(base, CH)], mb, semL)

    def process(ci, zb, mb, ub, vb, semG, semL, zb2, mb2, ub2, vb2, semG2, semL2):
        row = lax.rem(ci, SUP)
        base = wid * EPW + ci * CH
        c1 = ci + 1

        @pl.when(ci >= 1)
        def _drain_prev_scatter():
            pltpu.make_async_copy(mb2, agg_sp.at[tidxs.at[row]], semS).wait()

        @pl.when(jnp.logical_and(c1 < NCHUNK, lax.rem(c1, SUP) != 0))
        def _prefetch_early():
            fire(c1, zb2, mb2, ub2, vb2, semG2, semL2)

        pltpu.make_async_copy(z_hbm.at[sidxs.at[row]], zb, semG).wait()
        pltpu.make_async_copy(up_hbm.at[sidxs.at[row]], ub, semG).wait()
        pltpu.make_async_copy(vp_hbm.at[tidxs.at[row]], vb, semG).wait()
        pltpu.make_async_copy(me_hbm.at[pl.ds(base, CH)], mb, semL).wait()

        def relu_add(k, carry2):
            r = k >> 3
            col = (k & 7) * 16
            mv = mb[r, pl.ds(col, 16)]
            zv = zb[r, pl.ds(col, 16)]
            mb[r, pl.ds(col, 16)] = jnp.maximum(mv + zv, 0.0)
            return carry2
        lax.fori_loop(0, CH * (D // 16), relu_add, 0, unroll=4)

        pltpu.async_copy(mb, agg_sp.at[tidxs.at[row]], semS, add=True)

        @pl.when(ci >= 1)
        def _drain_prev_store():
            pltpu.make_async_copy(sbuf, enew_out.at[pl.ds(0, CH)], semT).wait()

        def snew(r, carry2):
            uv = ub[r, pl.ds(0, 16)]
            vv = vb[r, pl.ds(0, 16)]
            sbuf[r, pl.ds(0, 16)] = uv + vv
            return carry2
        lax.fori_loop(0, CH, snew, 0, unroll=4)
        pltpu.async_copy(sbuf, enew_out.at[pl.ds(base, CH)], semT)

        @pl.when(jnp.logical_and(c1 < NCHUNK, lax.rem(c1, SUP) == 0))
        def _prefetch_boundary():
            load_idx(c1 // SUP)
            fire(c1, zb2, mb2, ub2, vb2, semG2, semL2)

    pltpu.sync_copy(zro_hbm, zb128)
    zero_accum()
    plsc.subcore_barrier()

    load_idx(0)
    fire(0, zbufA, mebufA, ubufA, vbufA, semGA, semLA)

    def pair(i2, carry):
        ci = 2 * i2
        process(ci, zbufA, mebufA, ubufA, vbufA, semGA, semLA,
                zbufB, mebufB, ubufB, vbufB, semGB, semLB)
        process(ci + 1, zbufB, mebufB, ubufB, vbufB, semGB, semLB,
                zbufA, mebufA, ubufA, vbufA, semGA, semLA)
        return carry

    lax.fori_loop(0, NCHUNK // 2, pair, 0)
    pltpu.make_async_copy(mebufB, agg_sp.at[tidxs.at[0]], semS).wait()
    pltpu.make_async_copy(sbuf, enew_out.at[pl.ds(0, CH)], semT).wait()
    plsc.subcore_barrier()
    copy_accum_out(agg_out)
    plsc.subcore_barrier()

    pltpu.sync_copy(zro_hbm, zb128)
    zero_accum()
    pltpu.sync_copy(one_hbm, mebufA)
    plsc.subcore_barrier()

    def deg_block(b, carry):
        load_idx(b)
        for k in range(SUP):
            pltpu.async_copy(mebufA, agg_sp.at[tidxs.at[k]], semD, add=True)
        for k in range(SUP):
            pltpu.make_async_copy(mebufA, agg_sp.at[tidxs.at[k]], semD).wait()
        return carry

    lax.fori_loop(0, NBLK, deg_block, 0)
    plsc.subcore_barrier()
    copy_accum_out(deg_out)


def _sc_edge_stage(Z, Me, Up, Vp, src, tgt):
    zro = jnp.zeros((CPY, D), jnp.float32)
    one = jnp.ones((CH, D), jnp.float32)
    mesh = plsc.VectorSubcoreMesh(core_axis_name="c", subcore_axis_name="s")
    fn = pl.kernel(
        _sc_body,
        out_type=(
            jax.ShapeDtypeStruct((NC, N, D), jnp.float32),
            jax.ShapeDtypeStruct((NC, N, D), jnp.float32),
            jax.ShapeDtypeStruct((E, DE), jnp.float32),
        ),
        mesh=mesh,
        scratch_types=[
            pltpu.VMEM((SUP, CH), jnp.int32),
            pltpu.VMEM((SUP, CH), jnp.int32),
            pltpu.VMEM((CH, D), jnp.float32),
            pltpu.VMEM((CH, D), jnp.float32),
            pltpu.VMEM((CH, D), jnp.float32),
            pltpu.VMEM((CH, D), jnp.float32),
            pltpu.VMEM((CH, D), jnp.float32),
            pltpu.VMEM((CH, D), jnp.float32),
            pltpu.VMEM((CH, D), jnp.float32),
            pltpu.VMEM((CH, D), jnp.float32),
            pltpu.VMEM((CH, DE), jnp.float32),
            pltpu.VMEM((CPY, D), jnp.float32),
            pltpu.VMEM_SHARED((N, D), jnp.float32),
            pltpu.SemaphoreType.DMA,
            pltpu.SemaphoreType.DMA,
            pltpu.SemaphoreType.DMA,
            pltpu.SemaphoreType.DMA,
            pltpu.SemaphoreType.DMA,
            pltpu.SemaphoreType.DMA,
            pltpu.SemaphoreType.DMA,
        ],
    )
    src3 = src.reshape(NW, NBLK, SUP, CH)
    tgt3 = tgt.reshape(NW, NBLK, SUP, CH)
    return fn(Z, Me, Up, Vp, src3, tgt3, zro, one)


def kernel(h, e, edge_index, P_w, P_b, Q_w, Q_b, W_w, W_b):
    src = edge_index[0]
    tgt = edge_index[1]

    nb = N // 1000
    Z, Up, Vp = pl.pallas_call(
        _node_pre_body,
        grid=(nb,),
        in_specs=[
            pl.BlockSpec((1000, D), lambda i: (i, 0)),
            pl.BlockSpec((D, D), lambda i: (0, 0)),
            pl.BlockSpec((D, DE), lambda i: (0, 0)),
            pl.BlockSpec((D, DE), lambda i: (0, 0)),
        ],
        out_specs=[
            pl.BlockSpec((1000, D), lambda i: (i, 0)),
            pl.BlockSpec((1000, D), lambda i: (i, 0)),
            pl.BlockSpec((1000, D), lambda i: (i, 0)),
        ],
        out_shape=[
            jax.ShapeDtypeStruct((N, D), jnp.float32),
            jax.ShapeDtypeStruct((N, D), jnp.float32),
            jax.ShapeDtypeStruct((N, D), jnp.float32),
        ],
    )(h, P_w[:D], W_w[DE:DE + D], W_w[DE + D:])

    eb = E // 2000
    Me, Te = pl.pallas_call(
        _edge_pre_body,
        grid=(eb,),
        in_specs=[
            pl.BlockSpec((2000, DE), lambda i: (i, 0)),
            pl.BlockSpec((DE, D), lambda i: (0, 0)),
            pl.BlockSpec((1, D), lambda i: (0, 0)),
            pl.BlockSpec((DE, DE), lambda i: (0, 0)),
            pl.BlockSpec((1, DE), lambda i: (0, 0)),
        ],
        out_specs=[
            pl.BlockSpec((2000, D), lambda i: (i, 0)),
            pl.BlockSpec((2000, DE), lambda i: (i, 0)),
        ],
        out_shape=[
            jax.ShapeDtypeStruct((E, D), jnp.float32),
            jax.ShapeDtypeStruct((E, DE), jnp.float32),
        ],
    )(e, P_w[D:], P_b.reshape(1, D), W_w[:DE], W_b.reshape(1, DE))

    agg, degs, S = _sc_edge_stage(Z, Me, Up, Vp, src, tgt)

    e_new = pl.pallas_call(
        _edge_post_body,
        grid=(eb,),
        in_specs=[
            pl.BlockSpec((2000, DE), lambda i: (i, 0)),
            pl.BlockSpec((2000, DE), lambda i: (i, 0)),
        ],
        out_specs=pl.BlockSpec((2000, DE), lambda i: (i, 0)),
        out_shape=jax.ShapeDtypeStruct((E, DE), jnp.float32),
    )(Te, S)

    h_new = pl.pallas_call(
        _node_post_body,
        grid=(nb,),
        in_specs=[
            pl.BlockSpec((1000, D), lambda i: (i, 0)),
            pl.BlockSpec((NC, 1000, D), lambda i: (0, i, 0)),
            pl.BlockSpec((NC, 1000, D), lambda i: (0, i, 0)),
            pl.BlockSpec((D, D), lambda i: (0, 0)),
            pl.BlockSpec((D, D), lambda i: (0, 0)),
            pl.BlockSpec((1, D), lambda i: (0, 0)),
        ],
        out_specs=pl.BlockSpec((1000, D), lambda i: (i, 0)),
        out_shape=jax.ShapeDtypeStruct((N, D), jnp.float32),
    )(h, agg, degs, Q_w[:D], Q_w[D:], Q_b.reshape(1, D))

    return (h_new, e_new)


# speedup vs baseline: 1.8004x; 1.0327x over previous
import jax
import jax.numpy as jnp
from jax import lax
from jax.experimental import pallas as pl
from jax.experimental.pallas import tpu as pltpu
from jax.experimental.pallas import tpu_sc as plsc

N = 10000
E = 320000
D = 128
DE = 16

NC = 2
NS = 16
NW = NC * NS
EPW = E // NW
CH = 40
NCHUNK = EPW // CH
SUP = 5
NBLK = NCHUNK // SUP
SROW = 624
CPY = 16
TAILB = NS * SROW
TAIL = N - TAILB


def _node_pre_body(h_ref, pwh_ref, wu_ref, wv_ref, z_ref, up_ref, vp_ref):
    h = h_ref[...]
    z_ref[...] = jnp.dot(h, pwh_ref[...], preferred_element_type=jnp.float32)
    up_ref[:, :DE] = jnp.dot(h, wu_ref[...], preferred_element_type=jnp.float32)
    up_ref[:, DE:] = jnp.zeros((h.shape[0], D - DE), jnp.float32)
    vp_ref[:, :DE] = jnp.dot(h, wv_ref[...], preferred_element_type=jnp.float32)
    vp_ref[:, DE:] = jnp.zeros((h.shape[0], D - DE), jnp.float32)


def _edge_pre_body(e_ref, pwe_ref, pb_ref, wwe_ref, wb_ref, me_ref, te_ref):
    ev = e_ref[...]
    me_ref[...] = jnp.dot(ev, pwe_ref[...], preferred_element_type=jnp.float32) + pb_ref[...]
    te_ref[...] = jnp.dot(ev, wwe_ref[...], preferred_element_type=jnp.float32) + wb_ref[...]


def _edge_post_body(te_ref, s_ref, o_ref):
    o_ref[...] = jnp.maximum(te_ref[...] + s_ref[...], 0.0)


def _node_post_body(h_ref, agg_ref, deg_ref, qh_ref, qa_ref, qb_ref, o_ref):
    agg = agg_ref[0] + agg_ref[1]
    deg = deg_ref[0][:, 0:1] + deg_ref[1][:, 0:1]
    aggn = agg / deg
    acc = (jnp.dot(h_ref[...], qh_ref[...], preferred_element_type=jnp.float32)
           + jnp.dot(aggn, qa_ref[...], preferred_element_type=jnp.float32)
           + qb_ref[...])
    o_ref[...] = jnp.maximum(acc, 0.0)


def _sc_body(z_hbm, me_hbm, up_hbm, vp_hbm, src3_hbm, tgt3_hbm,
             zro_hbm, one_hbm,
             agg_out, deg_out, enew_out,
             sidxs, tidxs, zbufA, mebufA, ubufA, vbufA,
             zbufB, mebufB, ubufB, vbufB, sbuf,
             agg_sp, semGA, semLA, semGB, semLB, semD, semS, semT):
    c = lax.axis_index("c")
    s = lax.axis_index("s")
    wid = s * NC + c

    def zero_accum():
        pltpu.sync_copy(zro_hbm, agg_sp.at[pl.ds(s * SROW, SROW)])

        @pl.when(s == NS - 1)
        def _zero_tail():
            pltpu.sync_copy(zro_hbm.at[pl.ds(0, TAIL)], agg_sp.at[pl.ds(TAILB, TAIL)])

    def copy_accum_out(out_ref):
        pltpu.sync_copy(agg_sp.at[pl.ds(s * SROW, SROW)],
                        out_ref.at[c, pl.ds(s * SROW, SROW)])

        @pl.when(s == NS - 1)
        def _copy_tail():
            pltpu.sync_copy(agg_sp.at[pl.ds(TAILB, TAIL)],
                            out_ref.at[c, pl.ds(TAILB, TAIL)])

    def load_idx(b):
        pltpu.sync_copy(src3_hbm.at[wid, b], sidxs)
        pltpu.sync_copy(tgt3_hbm.at[wid, b], tidxs)

    def fire(c1, zb, mb, ub, vb, semG, semL):
        row = lax.rem(c1, SUP)
        base = wid * EPW + c1 * CH
        pltpu.async_copy(z_hbm.at[sidxs.at[row]], zb, semG)
        pltpu.async_copy(up_hbm.at[sidxs.at[row]], ub, semG)
        pltpu.async_copy(vp_hbm.at[tidxs.at[row]], vb, semG)
        pltpu.async_copy(me_hbm.at[pl.ds(base, CH)], mb, semL)

    def process(ci, zb, mb, ub, vb, semG, semL, zb2, mb2, ub2, vb2, semG2, semL2):
        row = lax.rem(ci, SUP)
        base = wid * EPW + ci * CH
        c1 = ci + 1

        @pl.when(ci >= 1)
        def _drain_prev_scatter():
            pltpu.make_async_copy(mb2, agg_sp.at[tidxs.at[row]], semS).wait()

        @pl.when(jnp.logical_and(c1 < NCHUNK, lax.rem(c1, SUP) != 0))
        def _prefetch_early():
            fire(c1, zb2, mb2, ub2, vb2, semG2, semL2)

        pltpu.make_async_copy(z_hbm.at[sidxs.at[row]], zb, semG).wait()
        pltpu.make_async_copy(up_hbm.at[sidxs.at[row]], ub, semG).wait()
        pltpu.make_async_copy(vp_hbm.at[tidxs.at[row]], vb, semG).wait()
        pltpu.make_async_copy(me_hbm.at[pl.ds(base, CH)], mb, semL).wait()

        def relu_add(r, carry2):
            for j in range(D // 16):
                col = j * 16
                mv = mb[r, pl.ds(col, 16)]
                zv = zb[r, pl.ds(col, 16)]
                mb[r, pl.ds(col, 16)] = jnp.maximum(mv + zv, 0.0)
            return carry2
        lax.fori_loop(0, CH, relu_add, 0, unroll=2)

        pltpu.async_copy(mb, agg_sp.at[tidxs.at[row]], semS, add=True)

        @pl.when(ci >= 1)
        def _drain_prev_store():
            pltpu.make_async_copy(sbuf, enew_out.at[pl.ds(0, CH)], semT).wait()

        def snew(r, carry2):
            uv = ub[r, pl.ds(0, 16)]
            vv = vb[r, pl.ds(0, 16)]
            sbuf[r, pl.ds(0, 16)] = uv + vv
            return carry2
        lax.fori_loop(0, CH, snew, 0, unroll=4)
        pltpu.async_copy(sbuf, enew_out.at[pl.ds(base, CH)], semT)

        @pl.when(jnp.logical_and(c1 < NCHUNK, lax.rem(c1, SUP) == 0))
        def _prefetch_boundary():
            load_idx(c1 // SUP)
            fire(c1, zb2, mb2, ub2, vb2, semG2, semL2)

    zero_accum()
    plsc.subcore_barrier()

    load_idx(0)
    fire(0, zbufA, mebufA, ubufA, vbufA, semGA, semLA)

    def pair(i2, carry):
        ci = 2 * i2
        process(ci, zbufA, mebufA, ubufA, vbufA, semGA, semLA,
                zbufB, mebufB, ubufB, vbufB, semGB, semLB)
        process(ci + 1, zbufB, mebufB, ubufB, vbufB, semGB, semLB,
                zbufA, mebufA, ubufA, vbufA, semGA, semLA)
        return carry

    lax.fori_loop(0, NCHUNK // 2, pair, 0)
    pltpu.make_async_copy(mebufB, agg_sp.at[tidxs.at[0]], semS).wait()
    pltpu.make_async_copy(sbuf, enew_out.at[pl.ds(0, CH)], semT).wait()
    plsc.subcore_barrier()
    copy_accum_out(agg_out)
    plsc.subcore_barrier()

    zero_accum()
    pltpu.sync_copy(one_hbm, mebufA)
    plsc.subcore_barrier()

    def deg_block(b, carry):
        pltpu.sync_copy(tgt3_hbm.at[wid, b], tidxs)
        for k in range(SUP):
            pltpu.async_copy(mebufA, agg_sp.at[tidxs.at[k]], semD, add=True)
        for k in range(SUP):
            pltpu.make_async_copy(mebufA, agg_sp.at[tidxs.at[k]], semD).wait()
        return carry

    lax.fori_loop(0, NBLK, deg_block, 0)
    plsc.subcore_barrier()
    copy_accum_out(deg_out)


def _sc_edge_stage(Z, Me, Up, Vp, src, tgt):
    zro = jnp.zeros((SROW, D), jnp.float32)
    one = jnp.ones((CH, D), jnp.float32)
    mesh = plsc.VectorSubcoreMesh(core_axis_name="c", subcore_axis_name="s")
    fn = pl.kernel(
        _sc_body,
        out_type=(
            jax.ShapeDtypeStruct((NC, N, D), jnp.float32),
            jax.ShapeDtypeStruct((NC, N, D), jnp.float32),
            jax.ShapeDtypeStruct((E, DE), jnp.float32),
        ),
        mesh=mesh,
        scratch_types=[
            pltpu.VMEM((SUP, CH), jnp.int32),
            pltpu.VMEM((SUP, CH), jnp.int32),
            pltpu.VMEM((CH, D), jnp.float32),
            pltpu.VMEM((CH, D), jnp.float32),
            pltpu.VMEM((CH, D), jnp.float32),
            pltpu.VMEM((CH, D), jnp.float32),
            pltpu.VMEM((CH, D), jnp.float32),
            pltpu.VMEM((CH, D), jnp.float32),
            pltpu.VMEM((CH, D), jnp.float32),
            pltpu.VMEM((CH, D), jnp.float32),
            pltpu.VMEM((CH, DE), jnp.float32),
            pltpu.VMEM_SHARED((N, D), jnp.float32),
            pltpu.SemaphoreType.DMA,
            pltpu.SemaphoreType.DMA,
            pltpu.SemaphoreType.DMA,
            pltpu.SemaphoreType.DMA,
            pltpu.SemaphoreType.DMA,
            pltpu.SemaphoreType.DMA,
            pltpu.SemaphoreType.DMA,
        ],
    )
    src3 = src.reshape(NW, NBLK, SUP, CH)
    tgt3 = tgt.reshape(NW, NBLK, SUP, CH)
    return fn(Z, Me, Up, Vp, src3, tgt3, zro, one)


def kernel(h, e, edge_index, P_w, P_b, Q_w, Q_b, W_w, W_b):
    src = edge_index[0]
    tgt = edge_index[1]

    nb = N // 1000
    Z, Up, Vp = pl.pallas_call(
        _node_pre_body,
        grid=(nb,),
        in_specs=[
            pl.BlockSpec((1000, D), lambda i: (i, 0)),
            pl.BlockSpec((D, D), lambda i: (0, 0)),
            pl.BlockSpec((D, DE), lambda i: (0, 0)),
            pl.BlockSpec((D, DE), lambda i: (0, 0)),
        ],
        out_specs=[
            pl.BlockSpec((1000, D), lambda i: (i, 0)),
            pl.BlockSpec((1000, D), lambda i: (i, 0)),
            pl.BlockSpec((1000, D), lambda i: (i, 0)),
        ],
        out_shape=[
            jax.ShapeDtypeStruct((N, D), jnp.float32),
            jax.ShapeDtypeStruct((N, D), jnp.float32),
            jax.ShapeDtypeStruct((N, D), jnp.float32),
        ],
    )(h, P_w[:D], W_w[DE:DE + D], W_w[DE + D:])

    eb = E // 2000
    Me, Te = pl.pallas_call(
        _edge_pre_body,
        grid=(eb,),
        in_specs=[
            pl.BlockSpec((2000, DE), lambda i: (i, 0)),
            pl.BlockSpec((DE, D), lambda i: (0, 0)),
            pl.BlockSpec((1, D), lambda i: (0, 0)),
            pl.BlockSpec((DE, DE), lambda i: (0, 0)),
            pl.BlockSpec((1, DE), lambda i: (0, 0)),
        ],
        out_specs=[
            pl.BlockSpec((2000, D), lambda i: (i, 0)),
            pl.BlockSpec((2000, DE), lambda i: (i, 0)),
        ],
        out_shape=[
            jax.ShapeDtypeStruct((E, D), jnp.float32),
            jax.ShapeDtypeStruct((E, DE), jnp.float32),
        ],
    )(e, P_w[D:], P_b.reshape(1, D), W_w[:DE], W_b.reshape(1, DE))

    agg, degs, S = _sc_edge_stage(Z, Me, Up, Vp, src, tgt)

    e_new = pl.pallas_call(
        _edge_post_body,
        grid=(eb,),
        in_specs=[
            pl.BlockSpec((2000, DE), lambda i: (i, 0)),
            pl.BlockSpec((2000, DE), lambda i: (i, 0)),
        ],
        out_specs=pl.BlockSpec((2000, DE), lambda i: (i, 0)),
        out_shape=jax.ShapeDtypeStruct((E, DE), jnp.float32),
    )(Te, S)

    h_new = pl.pallas_call(
        _node_post_body,
        grid=(nb,),
        in_specs=[
            pl.BlockSpec((1000, D), lambda i: (i, 0)),
            pl.BlockSpec((NC, 1000, D), lambda i: (0, i, 0)),
            pl.BlockSpec((NC, 1000, D), lambda i: (0, i, 0)),
            pl.BlockSpec((D, D), lambda i: (0, 0)),
            pl.BlockSpec((D, D), lambda i: (0, 0)),
            pl.BlockSpec((1, D), lambda i: (0, 0)),
        ],
        out_specs=pl.BlockSpec((1000, D), lambda i: (i, 0)),
        out_shape=jax.ShapeDtypeStruct((N, D), jnp.float32),
    )(h, agg, degs, Q_w[:D], Q_w[D:], Q_b.reshape(1, D))

    return (h_new, e_new)


# speedup vs baseline: 1.9128x; 1.0625x over previous
import jax
import jax.numpy as jnp
from jax import lax
from jax.experimental import pallas as pl
from jax.experimental.pallas import tpu as pltpu
from jax.experimental.pallas import tpu_sc as plsc

N = 10000
E = 320000
D = 128
DE = 16

NC = 2
NS = 16
NW = NC * NS
EPW = E // NW
CH = 40
NCHUNK = EPW // CH
SUP = 10
NBLK = NCHUNK // SUP
SROW = 624
CPY = 16
TAILB = NS * SROW
TAIL = N - TAILB


def _node_pre_body(h_ref, pwh_ref, wu_ref, wv_ref, zu_ref, vp_ref):
    h = h_ref[...]
    zu_ref[:, :D] = jnp.dot(h, pwh_ref[...], preferred_element_type=jnp.float32)
    zu_ref[:, D:D + DE] = jnp.dot(h, wu_ref[...], preferred_element_type=jnp.float32)
    zu_ref[:, D + DE:] = jnp.zeros((h.shape[0], D - DE), jnp.float32)
    vp_ref[:, :DE] = jnp.dot(h, wv_ref[...], preferred_element_type=jnp.float32)
    vp_ref[:, DE:] = jnp.zeros((h.shape[0], D - DE), jnp.float32)


def _edge_pre_body(e_ref, pwe_ref, pb_ref, wwe_ref, wb_ref, me_ref, te_ref):
    ev = e_ref[...]
    me_ref[...] = jnp.dot(ev, pwe_ref[...], preferred_element_type=jnp.float32) + pb_ref[...]
    te_ref[...] = jnp.dot(ev, wwe_ref[...], preferred_element_type=jnp.float32) + wb_ref[...]


def _edge_post_body(te_ref, s_ref, o_ref):
    o_ref[...] = jnp.maximum(te_ref[...] + s_ref[...], 0.0)


def _node_post_body(h_ref, agg_ref, deg_ref, qh_ref, qa_ref, qb_ref, o_ref):
    agg = agg_ref[0] + agg_ref[1]
    deg = deg_ref[0][:, 0:1] + deg_ref[1][:, 0:1]
    aggn = agg / deg
    acc = (jnp.dot(h_ref[...], qh_ref[...], preferred_element_type=jnp.float32)
           + jnp.dot(aggn, qa_ref[...], preferred_element_type=jnp.float32)
           + qb_ref[...])
    o_ref[...] = jnp.maximum(acc, 0.0)


def _sc_body(zu_hbm, me_hbm, vp_hbm, src3_hbm, tgt3_hbm,
             zro_hbm, one_hbm,
             agg_out, deg_out, enew_out,
             sidxs, tidxs, zubufA, mebufA, vbufA,
             zubufB, mebufB, vbufB, sbuf,
             agg_sp, semGA, semLA, semGB, semLB, semD, semS, semT):
    c = lax.axis_index("c")
    s = lax.axis_index("s")
    wid = s * NC + c

    def zero_accum():
        pltpu.sync_copy(zro_hbm, agg_sp.at[pl.ds(s * SROW, SROW)])

        @pl.when(s == NS - 1)
        def _zero_tail():
            pltpu.sync_copy(zro_hbm.at[pl.ds(0, TAIL)], agg_sp.at[pl.ds(TAILB, TAIL)])

    def copy_accum_out(out_ref):
        pltpu.sync_copy(agg_sp.at[pl.ds(s * SROW, SROW)],
                        out_ref.at[c, pl.ds(s * SROW, SROW)])

        @pl.when(s == NS - 1)
        def _copy_tail():
            pltpu.sync_copy(agg_sp.at[pl.ds(TAILB, TAIL)],
                            out_ref.at[c, pl.ds(TAILB, TAIL)])

    def load_idx(b):
        pltpu.sync_copy(src3_hbm.at[wid, b], sidxs)
        pltpu.sync_copy(tgt3_hbm.at[wid, b], tidxs)

    def fire(c1, zb, mb, vb, semG, semL):
        row = lax.rem(c1, SUP)
        base = wid * EPW + c1 * CH
        pltpu.async_copy(zu_hbm.at[sidxs.at[row]], zb, semG)
        pltpu.async_copy(vp_hbm.at[tidxs.at[row]], vb, semG)
        pltpu.async_copy(me_hbm.at[pl.ds(base, CH)], mb, semL)

    def process(ci, zb, mb, vb, semG, semL, zb2, mb2, vb2, semG2, semL2):
        row = lax.rem(ci, SUP)
        base = wid * EPW + ci * CH
        c1 = ci + 1

        @pl.when(ci >= 1)
        def _drain_prev_scatter():
            pltpu.make_async_copy(mb2, agg_sp.at[tidxs.at[row]], semS).wait()

        @pl.when(jnp.logical_and(c1 < NCHUNK, lax.rem(c1, SUP) != 0))
        def _prefetch_early():
            fire(c1, zb2, mb2, vb2, semG2, semL2)

        pltpu.make_async_copy(zu_hbm.at[sidxs.at[row]], zb, semG).wait()
        pltpu.make_async_copy(vp_hbm.at[tidxs.at[row]], vb, semG).wait()
        pltpu.make_async_copy(me_hbm.at[pl.ds(base, CH)], mb, semL).wait()

        def relu_add(r, carry2):
            for j in range(D // 16):
                col = j * 16
                mv = mb[r, pl.ds(col, 16)]
                zv = zb[r, pl.ds(col, 16)]
                mb[r, pl.ds(col, 16)] = jnp.maximum(mv + zv, 0.0)
            return carry2
        lax.fori_loop(0, CH, relu_add, 0, unroll=2)

        pltpu.async_copy(mb, agg_sp.at[tidxs.at[row]], semS, add=True)

        @pl.when(ci >= 1)
        def _drain_prev_store():
            pltpu.make_async_copy(sbuf, enew_out.at[pl.ds(0, CH)], semT).wait()

        def snew(r, carry2):
            uv = zb[r, pl.ds(D, 16)]
            vv = vb[r, pl.ds(0, 16)]
            sbuf[r, pl.ds(0, 16)] = uv + vv
            return carry2
        lax.fori_loop(0, CH, snew, 0, unroll=4)
        pltpu.async_copy(sbuf, enew_out.at[pl.ds(base, CH)], semT)

        @pl.when(jnp.logical_and(c1 < NCHUNK, lax.rem(c1, SUP) == 0))
        def _prefetch_boundary():
            load_idx(c1 // SUP)
            fire(c1, zb2, mb2, vb2, semG2, semL2)

    zero_accum()
    plsc.subcore_barrier()

    load_idx(0)
    fire(0, zubufA, mebufA, vbufA, semGA, semLA)

    def pair(i2, carry):
        ci = 2 * i2
        process(ci, zubufA, mebufA, vbufA, semGA, semLA,
                zubufB, mebufB, vbufB, semGB, semLB)
        process(ci + 1, zubufB, mebufB, vbufB, semGB, semLB,
                zubufA, mebufA, vbufA, semGA, semLA)
        return carry

    lax.fori_loop(0, NCHUNK // 2, pair, 0)
    pltpu.make_async_copy(mebufB, agg_sp.at[tidxs.at[0]], semS).wait()
    pltpu.make_async_copy(sbuf, enew_out.at[pl.ds(0, CH)], semT).wait()
    plsc.subcore_barrier()
    copy_accum_out(agg_out)
    plsc.subcore_barrier()

    zero_accum()
    pltpu.sync_copy(one_hbm, mebufA)
    plsc.subcore_barrier()

    def deg_block(b, carry):
        pltpu.sync_copy(tgt3_hbm.at[wid, b], tidxs)
        for k in range(SUP):
            pltpu.async_copy(mebufA, agg_sp.at[tidxs.at[k]], semD, add=True)
        for k in range(SUP):
            pltpu.make_async_copy(mebufA, agg_sp.at[tidxs.at[k]], semD).wait()
        return carry

    lax.fori_loop(0, NBLK, deg_block, 0)
    plsc.subcore_barrier()
    copy_accum_out(deg_out)


def _sc_edge_stage(ZU, Me, Vp, src, tgt):
    zro = jnp.zeros((SROW, D), jnp.float32)
    one = jnp.ones((CH, D), jnp.float32)
    mesh = plsc.VectorSubcoreMesh(core_axis_name="c", subcore_axis_name="s")
    fn = pl.kernel(
        _sc_body,
        out_type=(
            jax.ShapeDtypeStruct((NC, N, D), jnp.float32),
            jax.ShapeDtypeStruct((NC, N, D), jnp.float32),
            jax.ShapeDtypeStruct((E, DE), jnp.float32),
        ),
        mesh=mesh,
        scratch_types=[
            pltpu.VMEM((SUP, CH), jnp.int32),
            pltpu.VMEM((SUP, CH), jnp.int32),
            pltpu.VMEM((CH, 2 * D), jnp.float32),
            pltpu.VMEM((CH, D), jnp.float32),
            pltpu.VMEM((CH, D), jnp.float32),
            pltpu.VMEM((CH, 2 * D), jnp.float32),
            pltpu.VMEM((CH, D), jnp.float32),
            pltpu.VMEM((CH, D), jnp.float32),
            pltpu.VMEM((CH, DE), jnp.float32),
            pltpu.VMEM_SHARED((N, D), jnp.float32),
            pltpu.SemaphoreType.DMA,
            pltpu.SemaphoreType.DMA,
            pltpu.SemaphoreType.DMA,
            pltpu.SemaphoreType.DMA,
            pltpu.SemaphoreType.DMA,
            pltpu.SemaphoreType.DMA,
            pltpu.SemaphoreType.DMA,
        ],
    )
    src3 = src.reshape(NW, NBLK, SUP, CH)
    tgt3 = tgt.reshape(NW, NBLK, SUP, CH)
    return fn(ZU, Me, Vp, src3, tgt3, zro, one)


def kernel(h, e, edge_index, P_w, P_b, Q_w, Q_b, W_w, W_b):
    src = edge_index[0]
    tgt = edge_index[1]

    nb = N // 1000
    ZU, Vp = pl.pallas_call(
        _node_pre_body,
        grid=(nb,),
        in_specs=[
            pl.BlockSpec((1000, D), lambda i: (i, 0)),
            pl.BlockSpec((D, D), lambda i: (0, 0)),
            pl.BlockSpec((D, DE), lambda i: (0, 0)),
            pl.BlockSpec((D, DE), lambda i: (0, 0)),
        ],
        out_specs=[
            pl.BlockSpec((1000, 2 * D), lambda i: (i, 0)),
            pl.BlockSpec((1000, D), lambda i: (i, 0)),
        ],
        out_shape=[
            jax.ShapeDtypeStruct((N, 2 * D), jnp.float32),
            jax.ShapeDtypeStruct((N, D), jnp.float32),
        ],
    )(h, P_w[:D], W_w[DE:DE + D], W_w[DE + D:])

    eb = E // 2000
    Me, Te = pl.pallas_call(
        _edge_pre_body,
        grid=(eb,),
        in_specs=[
            pl.BlockSpec((2000, DE), lambda i: (i, 0)),
            pl.BlockSpec((DE, D), lambda i: (0, 0)),
            pl.BlockSpec((1, D), lambda i: (0, 0)),
            pl.BlockSpec((DE, DE), lambda i: (0, 0)),
            pl.BlockSpec((1, DE), lambda i: (0, 0)),
        ],
        out_specs=[
            pl.BlockSpec((2000, D), lambda i: (i, 0)),
            pl.BlockSpec((2000, DE), lambda i: (i, 0)),
        ],
        out_shape=[
            jax.ShapeDtypeStruct((E, D), jnp.float32),
            jax.ShapeDtypeStruct((E, DE), jnp.float32),
        ],
    )(e, P_w[D:], P_b.reshape(1, D), W_w[:DE], W_b.reshape(1, DE))

    agg, degs, S = _sc_edge_stage(ZU, Me, Vp, src, tgt)

    e_new = pl.pallas_call(
        _edge_post_body,
        grid=(eb,),
        in_specs=[
            pl.BlockSpec((2000, DE), lambda i: (i, 0)),
            pl.BlockSpec((2000, DE), lambda i: (i, 0)),
        ],
        out_specs=pl.BlockSpec((2000, DE), lambda i: (i, 0)),
        out_shape=jax.ShapeDtypeStruct((E, DE), jnp.float32),
    )(Te, S)

    h_new = pl.pallas_call(
        _node_post_body,
        grid=(nb,),
        in_specs=[
            pl.BlockSpec((1000, D), lambda i: (i, 0)),
            pl.BlockSpec((NC, 1000, D), lambda i: (0, i, 0)),
            pl.BlockSpec((NC, 1000, D), lambda i: (0, i, 0)),
            pl.BlockSpec((D, D), lambda i: (0, 0)),
            pl.BlockSpec((D, D), lambda i: (0, 0)),
            pl.BlockSpec((1, D), lambda i: (0, 0)),
        ],
        out_specs=pl.BlockSpec((1000, D), lambda i: (i, 0)),
        out_shape=jax.ShapeDtypeStruct((N, D), jnp.float32),
    )(h, agg, degs, Q_w[:D], Q_w[D:], Q_b.reshape(1, D))

    return (h_new, e_new)
